# Initial kernel scaffold; baseline (speedup 1.0000x reference)
import jax
import jax.numpy as jnp
from jax.experimental import pallas as pl


def kernel(user_idx, item_idx, user_tables, item_tables, Wu1, bu1, Wu2, bu2, Wi1, bi1, Wi2, bi2):
    raise NotImplementedError("write your pallas kernel here")


# speedup vs baseline: 8.4076x; 8.4076x over previous
import functools

import jax
import jax.numpy as jnp
from jax import lax
from jax.experimental import pallas as pl
from jax.experimental.pallas import tpu as pltpu
from jax.experimental.pallas import tpu_sc as plsc

F = 13
V = 100000
E = 32
B = 16384
D_IN = F * E
H1 = 256
H2 = 128
EPS = 1e-12

TOT = B * F
NW = 32
PW = TOT // NW
CH = 1664
NCH = PW // CH


def _sc_gather_fn():
    mesh = plsc.VectorSubcoreMesh(core_axis_name="c", subcore_axis_name="s")

    @functools.partial(
        pl.kernel,
        mesh=mesh,
        out_type=[
            jax.ShapeDtypeStruct((TOT, E), jnp.float32),
            jax.ShapeDtypeStruct((TOT, E), jnp.float32),
        ],
        scratch_types=[
            pltpu.VMEM((CH,), jnp.int32),
            pltpu.VMEM((CH, E), jnp.float32),
            pltpu.SemaphoreType.DMA,
        ],
        compiler_params=pltpu.CompilerParams(use_tc_tiling_on_sc=False),
    )
    def gather_k(uidx_hbm, iidx_hbm, utab_hbm, itab_hbm,
                 uout_hbm, iout_hbm, idx_v, rows_v, sem):
        wid = lax.axis_index("s") * 2 + lax.axis_index("c")
        for idx_hbm, tab_hbm, out_hbm in (
            (uidx_hbm, utab_hbm, uout_hbm),
            (iidx_hbm, itab_hbm, iout_hbm),
        ):
            for c in range(NCH):
                base = wid * PW + c * CH
                pltpu.sync_copy(idx_hbm.at[pl.ds(base, CH)], idx_v)
                pltpu.async_copy(tab_hbm.at[idx_v], rows_v, sem).wait()
                pltpu.sync_copy(rows_v, out_hbm.at[pl.ds(base, CH)])

    return gather_k


_SC_GATHER = _sc_gather_fn()

BB = 512


def _tc_body(xu_ref, xi_ref, wu1_ref, bu1_ref, wu2_ref, bu2_ref,
             wi1_ref, bi1_ref, wi2_ref, bi2_ref, out_ref):
    xu = xu_ref[...]
    hu = jnp.maximum(
        jnp.dot(xu, wu1_ref[...], preferred_element_type=jnp.float32)
        + bu1_ref[...], 0.0)
    hu = jnp.maximum(
        jnp.dot(hu, wu2_ref[...], preferred_element_type=jnp.float32)
        + bu2_ref[...], 0.0)
    xi = xi_ref[...]
    hi = jnp.maximum(
        jnp.dot(xi, wi1_ref[...], preferred_element_type=jnp.float32)
        + bi1_ref[...], 0.0)
    hi = jnp.maximum(
        jnp.dot(hi, wi2_ref[...], preferred_element_type=jnp.float32)
        + bi2_ref[...], 0.0)
    dot = jnp.sum(hu * hi, axis=1)
    nu = jnp.sum(hu * hu, axis=1)
    ni = jnp.sum(hi * hi, axis=1)
    denom = jnp.maximum(jnp.sqrt(nu), EPS) * jnp.maximum(jnp.sqrt(ni), EPS)
    out_ref[...] = jax.nn.sigmoid(dot / denom)


def _tc_forward(xu, xi, Wu1, bu1, Wu2, bu2, Wi1, bi1, Wi2, bi2):
    full = lambda shape: pl.BlockSpec(shape, lambda i: (0,) * len(shape))
    return pl.pallas_call(
        _tc_body,
        grid=(B // BB,),
        in_specs=[
            pl.BlockSpec((BB, D_IN), lambda i: (i, 0)),
            pl.BlockSpec((BB, D_IN), lambda i: (i, 0)),
            full((D_IN, H1)), full((1, H1)), full((H1, H2)), full((1, H2)),
            full((D_IN, H1)), full((1, H1)), full((H1, H2)), full((1, H2)),
        ],
        out_specs=pl.BlockSpec((BB,), lambda i: (i,)),
        out_shape=jax.ShapeDtypeStruct((B,), jnp.float32),
    )(xu, xi, Wu1, bu1, Wu2, bu2, Wi1, bi1, Wi2, bi2)


def kernel(user_idx, item_idx, user_tables, item_tables,
           Wu1, bu1, Wu2, bu2, Wi1, bi1, Wi2, bi2):
    offs = (jnp.arange(F, dtype=jnp.int32) * V)[None, :]
    uflat = (user_idx.astype(jnp.int32) + offs).reshape(-1)
    iflat = (item_idx.astype(jnp.int32) + offs).reshape(-1)
    utab = user_tables.reshape(F * V, E)
    itab = item_tables.reshape(F * V, E)
    urows, irows = _SC_GATHER(uflat, iflat, utab, itab)
    xu = urows.reshape(B, D_IN)
    xi = irows.reshape(B, D_IN)
    return _tc_forward(xu, xi, Wu1, bu1.reshape(1, H1), Wu2, bu2.reshape(1, H2),
                       Wi1, bi1.reshape(1, H1), Wi2, bi2.reshape(1, H2))


# speedup vs baseline: 8.4841x; 1.0091x over previous
import functools

import jax
import jax.numpy as jnp
from jax import lax
from jax.experimental import pallas as pl
from jax.experimental.pallas import tpu as pltpu
from jax.experimental.pallas import tpu_sc as plsc

F = 13
V = 100000
E = 32
B = 16384
H1 = 256
H2 = 128
EPS = 1e-12

TOT = B * F
NW = 32
PW = TOT // NW
CH = 1664
NCH = PW // CH

K1CV = 3200
K1NB = 32


def _k1_body(x_ref, o_ref):
    x = x_ref[0]
    t = jnp.transpose(x, (1, 0))
    t3 = t.reshape(K1CV // 4, 4, E)
    for q in range(4):
        o_ref[0, :, q * E:(q + 1) * E] = t3[:, q, :]


def _k1_transpose(tab3):
    return pl.pallas_call(
        _k1_body,
        grid=(F, K1NB),
        in_specs=[pl.BlockSpec((1, 32, K1CV), lambda f, k: (f, 0, k))],
        out_specs=pl.BlockSpec((1, K1CV // 4, 128), lambda f, k: (f, k, 0)),
        out_shape=jax.ShapeDtypeStruct((F, V // 4, 128), jnp.float32),
    )(tab3)


def _k2_gather_fn():
    mesh = plsc.VectorSubcoreMesh(core_axis_name="c", subcore_axis_name="s")

    @functools.partial(
        pl.kernel,
        mesh=mesh,
        out_type=jax.ShapeDtypeStruct((TOT, E), jnp.float32),
        scratch_types=[
            pltpu.VMEM((CH,), jnp.int32),
            pltpu.VMEM((CH, E), jnp.float32),
            pltpu.SemaphoreType.DMA,
        ],
        compiler_params=pltpu.CompilerParams(use_tc_tiling_on_sc=False),
    )
    def k2(idx_hbm, tab_hbm, out_hbm, idx_v, rows_v, sem):
        wid = lax.axis_index("s") * 2 + lax.axis_index("c")
        for c in range(NCH):
            base = wid * PW + c * CH
            pltpu.sync_copy(idx_hbm.at[pl.ds(base, CH)], idx_v)
            pltpu.async_copy(tab_hbm.at[idx_v], rows_v, sem).wait()
            pltpu.sync_copy(rows_v, out_hbm.at[pl.ds(base, CH)])

    return k2


_K2 = _k2_gather_fn()

BBG = 512
NBLK = (B // 4) // BBG


def _k3_body(xu_ref, xi_ref, w1u_ref, w1i_ref, w2u_ref, w2i_ref,
             b1u_ref, b1i_ref, b2u_ref, b2i_ref, m_ref, out_ref,
             hu_ref, hi_ref):
    p = pl.program_id(1)

    @pl.when(p == 0)
    def _():
        hu_ref[...] = jnp.broadcast_to(b1u_ref[...], (BBG, 4 * H1))
        hi_ref[...] = jnp.broadcast_to(b1i_ref[...], (BBG, 4 * H1))

    xu = xu_ref[0].astype(jnp.bfloat16)
    xi = xi_ref[0].astype(jnp.bfloat16)
    hu_ref[...] += jnp.dot(xu, w1u_ref[0],
                           preferred_element_type=jnp.float32)
    hi_ref[...] += jnp.dot(xi, w1i_ref[0],
                           preferred_element_type=jnp.float32)

    @pl.when(p == F - 1)
    def _():
        hu = jnp.maximum(hu_ref[...], 0.0).astype(jnp.bfloat16)
        hi = jnp.maximum(hi_ref[...], 0.0).astype(jnp.bfloat16)
        zu = jnp.maximum(
            jnp.dot(hu, w2u_ref[...], preferred_element_type=jnp.float32)
            + b2u_ref[...], 0.0)
        zi = jnp.maximum(
            jnp.dot(hi, w2i_ref[...], preferred_element_type=jnp.float32)
            + b2i_ref[...], 0.0)
        m = m_ref[...]
        dots = jnp.dot(zu * zi, m, preferred_element_type=jnp.float32)
        nu = jnp.dot(zu * zu, m, preferred_element_type=jnp.float32)
        ni = jnp.dot(zi * zi, m, preferred_element_type=jnp.float32)
        den = (jnp.maximum(jnp.sqrt(nu), EPS)
               * jnp.maximum(jnp.sqrt(ni), EPS))
        out_ref[...] = jax.nn.sigmoid(dots / den)


def _k3(xu3, xi3, w1u, w1i, w2u, w2i, b1u, b1i, b2u, b2i, m):
    full = lambda shape: pl.BlockSpec(shape, lambda i, p: (0,) * len(shape))
    return pl.pallas_call(
        _k3_body,
        grid=(NBLK, F),
        in_specs=[
            pl.BlockSpec((1, BBG, 128), lambda i, p: (p, i, 0)),
            pl.BlockSpec((1, BBG, 128), lambda i, p: (p, i, 0)),
            pl.BlockSpec((1, 128, 4 * H1), lambda i, p: (p, 0, 0)),
            pl.BlockSpec((1, 128, 4 * H1), lambda i, p: (p, 0, 0)),
            full((4 * H1, 4 * H2)), full((4 * H1, 4 * H2)),
            full((1, 4 * H1)), full((1, 4 * H1)),
            full((1, 4 * H2)), full((1, 4 * H2)),
            full((4 * H2, 4)),
        ],
        out_specs=pl.BlockSpec((BBG, 4), lambda i, p: (i, 0)),
        out_shape=jax.ShapeDtypeStruct((B // 4, 4), jnp.float32),
        scratch_shapes=[
            pltpu.VMEM((BBG, 4 * H1), jnp.float32),
            pltpu.VMEM((BBG, 4 * H1), jnp.float32),
        ],
    )(xu3, xi3, w1u, w1i, w2u, w2i, b1u, b1i, b2u, b2i, m)


def _blockdiag_w1(w1):
    w1r = w1.reshape(F, E, H1)
    out = jnp.zeros((F, 4, E, 4, H1), w1.dtype)
    for q in range(4):
        out = out.at[:, q, :, q, :].set(w1r)
    return out.reshape(F, 4 * E, 4 * H1).astype(jnp.bfloat16)


def _blockdiag_w2(w2):
    out = jnp.zeros((4, H1, 4, H2), w2.dtype)
    for q in range(4):
        out = out.at[q, :, q, :].set(w2)
    return out.reshape(4 * H1, 4 * H2).astype(jnp.bfloat16)


def kernel(user_idx, item_idx, user_tables, item_tables,
           Wu1, bu1, Wu2, bu2, Wi1, bi1, Wi2, bi2):
    ut3 = user_tables.transpose(0, 2, 1)
    it3 = item_tables.transpose(0, 2, 1)
    offs = (jnp.arange(F, dtype=jnp.int32) * V)[:, None]
    uidx = (user_idx.T + offs).reshape(-1)
    iidx = (item_idx.T + offs).reshape(-1)

    tu = _k1_transpose(ut3)
    xu = _K2(uidx, tu.reshape(F * V, E))
    ti = _k1_transpose(it3)
    xi = _K2(iidx, ti.reshape(F * V, E))
    xu3 = xu.reshape(F, B // 4, 128)
    xi3 = xi.reshape(F, B // 4, 128)

    m = jnp.kron(jnp.eye(4, dtype=jnp.float32),
                 jnp.ones((H2, 1), jnp.float32))
    y4 = _k3(xu3, xi3,
             _blockdiag_w1(Wu1), _blockdiag_w1(Wi1),
             _blockdiag_w2(Wu2), _blockdiag_w2(Wi2),
             jnp.tile(bu1, 4)[None], jnp.tile(bi1, 4)[None],
             jnp.tile(bu2, 4)[None], jnp.tile(bi2, 4)[None], m)
    return y4.reshape(B)


# speedup vs baseline: 9.0136x; 1.0624x over previous
import functools

import jax
import jax.numpy as jnp
from jax import lax
from jax.experimental import pallas as pl
from jax.experimental.pallas import tpu as pltpu
from jax.experimental.pallas import tpu_sc as plsc

F = 13
V = 100000
E = 32
B = 16384
H1 = 256
H2 = 128
EPS = 1e-12

TOT = B * F
NW = 32
PW = TOT // NW
CH = 1664
NCH = PW // CH

K1CV = 3200
K1NB = 32


def _k1_body(x_ref, o_ref):
    x = x_ref[0]
    t = jnp.transpose(x, (1, 0))
    t3 = t.reshape(K1CV // 4, 4, E)
    for q in range(4):
        o_ref[0, :, q * E:(q + 1) * E] = t3[:, q, :]


def _k1_transpose(tab3):
    return pl.pallas_call(
        _k1_body,
        grid=(F, K1NB),
        in_specs=[pl.BlockSpec((1, 32, K1CV), lambda f, k: (f, 0, k))],
        out_specs=pl.BlockSpec((1, K1CV // 4, 128), lambda f, k: (f, k, 0)),
        out_shape=jax.ShapeDtypeStruct((F, V // 4, 128), jnp.float32),
        compiler_params=pltpu.CompilerParams(
            fuse_transposed_lhs_in_matmul=True),
    )(tab3)


def _k2_gather_fn():
    mesh = plsc.VectorSubcoreMesh(core_axis_name="c", subcore_axis_name="s")

    @functools.partial(
        pl.kernel,
        mesh=mesh,
        out_type=jax.ShapeDtypeStruct((TOT, E), jnp.float32),
        scratch_types=[
            pltpu.VMEM((CH,), jnp.int32),
            pltpu.VMEM((CH, E), jnp.float32),
            pltpu.SemaphoreType.DMA,
        ],
        compiler_params=pltpu.CompilerParams(use_tc_tiling_on_sc=False),
    )
    def k2(idx_hbm, tab_hbm, out_hbm, idx_v, rows_v, sem):
        wid = lax.axis_index("s") * 2 + lax.axis_index("c")
        for c in range(NCH):
            base = wid * PW + c * CH
            pltpu.sync_copy(idx_hbm.at[pl.ds(base, CH)], idx_v)
            pltpu.async_copy(tab_hbm.at[idx_v], rows_v, sem).wait()
            pltpu.sync_copy(rows_v, out_hbm.at[pl.ds(base, CH)])

    return k2


_K2 = _k2_gather_fn()

BBG = 512
NBLK = (B // 4) // BBG


def _k3_body(xu_ref, xi_ref, w1u_ref, w1i_ref, w2u_ref, w2i_ref,
             b1u_ref, b1i_ref, b2u_ref, b2i_ref, m_ref, out_ref):
    xu = jnp.concatenate([xu_ref[p].astype(jnp.bfloat16) for p in range(F)],
                         axis=1)
    xi = jnp.concatenate([xi_ref[p].astype(jnp.bfloat16) for p in range(F)],
                         axis=1)
    hu = jnp.maximum(
        jnp.dot(xu, w1u_ref[...], preferred_element_type=jnp.float32)
        + b1u_ref[...], 0.0).astype(jnp.bfloat16)
    hi = jnp.maximum(
        jnp.dot(xi, w1i_ref[...], preferred_element_type=jnp.float32)
        + b1i_ref[...], 0.0).astype(jnp.bfloat16)
    zu = jnp.maximum(
        jnp.dot(hu, w2u_ref[...], preferred_element_type=jnp.float32)
        + b2u_ref[...], 0.0)
    zi = jnp.maximum(
        jnp.dot(hi, w2i_ref[...], preferred_element_type=jnp.float32)
        + b2i_ref[...], 0.0)
    m = m_ref[...]
    dots = jnp.dot(zu * zi, m, preferred_element_type=jnp.float32)
    nu = jnp.dot(zu * zu, m, preferred_element_type=jnp.float32)
    ni = jnp.dot(zi * zi, m, preferred_element_type=jnp.float32)
    den = (jnp.maximum(jnp.sqrt(nu), EPS)
           * jnp.maximum(jnp.sqrt(ni), EPS))
    out_ref[...] = jax.nn.sigmoid(dots / den)


def _k3(xu3, xi3, w1u, w1i, w2u, w2i, b1u, b1i, b2u, b2i, m):
    full = lambda shape: pl.BlockSpec(shape, lambda i: (0,) * len(shape))
    return pl.pallas_call(
        _k3_body,
        grid=(NBLK,),
        in_specs=[
            pl.BlockSpec((F, BBG, 128), lambda i: (0, i, 0)),
            pl.BlockSpec((F, BBG, 128), lambda i: (0, i, 0)),
            full((F * 128, 4 * H1)), full((F * 128, 4 * H1)),
            full((4 * H1, 4 * H2)), full((4 * H1, 4 * H2)),
            full((1, 4 * H1)), full((1, 4 * H1)),
            full((1, 4 * H2)), full((1, 4 * H2)),
            full((4 * H2, 4)),
        ],
        out_specs=pl.BlockSpec((BBG, 4), lambda i: (i, 0)),
        out_shape=jax.ShapeDtypeStruct((B // 4, 4), jnp.float32),
    )(xu3, xi3, w1u, w1i, w2u, w2i, b1u, b1i, b2u, b2i, m)


def _blockdiag_w1(w1):
    w1r = w1.reshape(F, E, H1)
    out = jnp.zeros((F, 4, E, 4, H1), w1.dtype)
    for q in range(4):
        out = out.at[:, q, :, q, :].set(w1r)
    return out.reshape(F * 4 * E, 4 * H1).astype(jnp.bfloat16)


def _blockdiag_w2(w2):
    out = jnp.zeros((4, H1, 4, H2), w2.dtype)
    for q in range(4):
        out = out.at[q, :, q, :].set(w2)
    return out.reshape(4 * H1, 4 * H2).astype(jnp.bfloat16)


def kernel(user_idx, item_idx, user_tables, item_tables,
           Wu1, bu1, Wu2, bu2, Wi1, bi1, Wi2, bi2):
    ut3 = user_tables.transpose(0, 2, 1)
    it3 = item_tables.transpose(0, 2, 1)
    offs = (jnp.arange(F, dtype=jnp.int32) * V)[:, None]
    uidx = (user_idx.T + offs).reshape(-1)
    iidx = (item_idx.T + offs).reshape(-1)

    tu = _k1_transpose(ut3)
    xu = _K2(uidx, tu.reshape(F * V, E))
    ti = _k1_transpose(it3)
    xi = _K2(iidx, ti.reshape(F * V, E))
    xu3 = xu.reshape(F, B // 4, 128)
    xi3 = xi.reshape(F, B // 4, 128)

    m = jnp.kron(jnp.eye(4, dtype=jnp.float32),
                 jnp.ones((H2, 1), jnp.float32))
    y4 = _k3(xu3, xi3,
             _blockdiag_w1(Wu1), _blockdiag_w1(Wi1),
             _blockdiag_w2(Wu2), _blockdiag_w2(Wi2),
             jnp.tile(bu1, 4)[None], jnp.tile(bi1, 4)[None],
             jnp.tile(bu2, 4)[None], jnp.tile(bi2, 4)[None], m)
    return y4.reshape(B)
